# baseline (speedup 1.0000x reference)
import jax
import jax.numpy as jnp
from jax.experimental import pallas as pl
from jax.experimental.pallas import tpu as pltpu


def _add_kernel(ids_ref, gate_ref, hid_ref, emb_ref, out_ref):
    scale = jnp.tanh(gate_ref[0])
    out_ref[...] = hid_ref[...] + emb_ref[...] * scale


def kernel(hidden_state, aspect_ratio_ids, embedding_table, gate):
    b, t, p, h = hidden_state.shape
    table = embedding_table.reshape(-1, t, 1, h)
    out = pl.pallas_call(
        _add_kernel,
        grid_spec=pltpu.PrefetchScalarGridSpec(
            num_scalar_prefetch=2,
            grid=(b, t),
            in_specs=[
                pl.BlockSpec((1, 1, p, h), lambda i, j, ids, gate: (i, j, 0, 0)),
                pl.BlockSpec((1, 1, 1, h), lambda i, j, ids, gate: (ids[i], j, 0, 0)),
            ],
            out_specs=pl.BlockSpec((1, 1, p, h), lambda i, j, ids, gate: (i, j, 0, 0)),
        ),
        out_shape=jax.ShapeDtypeStruct(hidden_state.shape, hidden_state.dtype),
    )(aspect_ratio_ids, gate, hidden_state, table)
    return out


# speedup vs baseline: 1.0046x; 1.0046x over previous
import jax
import jax.numpy as jnp
from jax.experimental import pallas as pl
from jax.experimental.pallas import tpu as pltpu

_R = 4


def _stream_kernel(ids_ref, gate_ref, hid_ref, table_ref, out_ref,
                   in_buf, out_buf, emb_buf, in_sem, out_sem, emb_sem):
    nb, nt, p, h = hid_ref.shape
    n = nb * nt

    for bb in range(nb):
        pltpu.make_async_copy(
            table_ref.at[pl.ds(ids_ref[bb], 1)],
            emb_buf.at[pl.ds(bb, 1)], emb_sem).start()

    for i in range(min(_R, n)):
        b, t = divmod(i, nt)
        pltpu.make_async_copy(
            hid_ref.at[b, t], in_buf.at[i % _R], in_sem.at[i % _R]).start()

    for bb in range(nb):
        pltpu.make_async_copy(
            table_ref.at[pl.ds(ids_ref[bb], 1)],
            emb_buf.at[pl.ds(bb, 1)], emb_sem).wait()

    scale = jnp.tanh(gate_ref[0])

    for i in range(n):
        b, t = divmod(i, nt)
        r = i % _R
        pltpu.make_async_copy(hid_ref.at[b, t], in_buf.at[r], in_sem.at[r]).wait()
        if i >= _R:
            bo, to = divmod(i - _R, nt)
            pltpu.make_async_copy(
                out_buf.at[r], out_ref.at[bo, to], out_sem.at[r]).wait()
        emb = emb_buf[pl.ds(b, 1), pl.ds(t * h, h)]
        out_buf[r, :, :] = in_buf[r] + emb * scale
        pltpu.make_async_copy(out_buf.at[r], out_ref.at[b, t], out_sem.at[r]).start()
        j = i + _R
        if j < n:
            bj, tj = divmod(j, nt)
            pltpu.make_async_copy(
                hid_ref.at[bj, tj], in_buf.at[r], in_sem.at[r]).start()

    for i in range(max(0, n - _R), n):
        b, t = divmod(i, nt)
        pltpu.make_async_copy(
            out_buf.at[i % _R], out_ref.at[b, t], out_sem.at[i % _R]).wait()


def kernel(hidden_state, aspect_ratio_ids, embedding_table, gate):
    b, t, p, h = hidden_state.shape
    out = pl.pallas_call(
        _stream_kernel,
        in_specs=[
            pl.BlockSpec(memory_space=pltpu.SMEM),
            pl.BlockSpec(memory_space=pltpu.SMEM),
            pl.BlockSpec(memory_space=pl.ANY),
            pl.BlockSpec(memory_space=pl.ANY),
        ],
        out_specs=pl.BlockSpec(memory_space=pl.ANY),
        out_shape=jax.ShapeDtypeStruct(hidden_state.shape, hidden_state.dtype),
        scratch_shapes=[
            pltpu.VMEM((_R, p, h), jnp.float32),
            pltpu.VMEM((_R, p, h), jnp.float32),
            pltpu.VMEM((b, t * h), jnp.float32),
            pltpu.SemaphoreType.DMA((_R,)),
            pltpu.SemaphoreType.DMA((_R,)),
            pltpu.SemaphoreType.DMA,
        ],
    )(aspect_ratio_ids, gate, hidden_state, embedding_table)
    return out


# speedup vs baseline: 1.0056x; 1.0010x over previous
import jax
import jax.numpy as jnp
from jax.experimental import pallas as pl
from jax.experimental.pallas import tpu as pltpu

_R = 8
_CROWS = 256


def _stream_kernel(ids_ref, gate_ref, hid_ref, table_ref, out_ref,
                   in_buf, out_buf, emb_buf, in_sem, out_sem, emb_sem):
    nb, nt, p, h = hid_ref.shape

    chunks = []
    for b in range(nb):
        for t in range(nt):
            r0 = 0
            while r0 < p:
                rows = min(_CROWS, p - r0)
                chunks.append((b, t, r0, rows))
                r0 += rows
    n = len(chunks)

    def in_copy(c, r):
        b, t, r0, rows = chunks[c]
        return pltpu.make_async_copy(
            hid_ref.at[b, t, pl.ds(r0, rows), :],
            in_buf.at[r, pl.ds(0, rows), :], in_sem.at[r])

    def out_copy(c, r):
        b, t, r0, rows = chunks[c]
        return pltpu.make_async_copy(
            out_buf.at[r, pl.ds(0, rows), :],
            out_ref.at[b, t, pl.ds(r0, rows), :], out_sem.at[r])

    for bb in range(nb):
        pltpu.make_async_copy(
            table_ref.at[pl.ds(ids_ref[bb], 1)],
            emb_buf.at[pl.ds(bb, 1)], emb_sem).start()

    for c in range(min(_R, n)):
        in_copy(c, c % _R).start()

    for bb in range(nb):
        pltpu.make_async_copy(
            table_ref.at[pl.ds(ids_ref[bb], 1)],
            emb_buf.at[pl.ds(bb, 1)], emb_sem).wait()

    scale = jnp.tanh(gate_ref[0])

    for c in range(n):
        r = c % _R
        b, t, r0, rows = chunks[c]
        in_copy(c, r).wait()
        if c >= _R:
            out_copy(c - _R, r).wait()
        emb = emb_buf[pl.ds(b, 1), pl.ds(t * h, h)]
        out_buf[r, pl.ds(0, rows), :] = in_buf[r, pl.ds(0, rows), :] + emb * scale
        out_copy(c, r).start()
        if c + _R < n:
            in_copy(c + _R, r).start()

    for c in range(max(0, n - _R), n):
        out_copy(c, c % _R).wait()


def kernel(hidden_state, aspect_ratio_ids, embedding_table, gate):
    b, t, p, h = hidden_state.shape
    out = pl.pallas_call(
        _stream_kernel,
        in_specs=[
            pl.BlockSpec(memory_space=pltpu.SMEM),
            pl.BlockSpec(memory_space=pltpu.SMEM),
            pl.BlockSpec(memory_space=pl.ANY),
            pl.BlockSpec(memory_space=pl.ANY),
        ],
        out_specs=pl.BlockSpec(memory_space=pl.ANY),
        out_shape=jax.ShapeDtypeStruct(hidden_state.shape, hidden_state.dtype),
        scratch_shapes=[
            pltpu.VMEM((_R, _CROWS, h), jnp.float32),
            pltpu.VMEM((_R, _CROWS, h), jnp.float32),
            pltpu.VMEM((b, t * h), jnp.float32),
            pltpu.SemaphoreType.DMA((_R,)),
            pltpu.SemaphoreType.DMA((_R,)),
            pltpu.SemaphoreType.DMA,
        ],
    )(aspect_ratio_ids, gate, hidden_state, embedding_table)
    return out
